# baseline (speedup 1.0000x reference)
import jax
import jax.numpy as jnp
from jax.experimental import pallas as pl
from jax.experimental.pallas import tpu as pltpu

B = 4
CHUNK = 1024
NBUF = 2


def _dma_body(table_hbm, out_hbm, buf0, buf1, rsem0, rsem1, wsem0, wsem1):
    S = out_hbm.shape[1]
    nchunk = S // CHUNK
    bufs = (buf0, buf1)
    rsems = (rsem0, rsem1)
    wsems = (wsem0, wsem1)

    reads = [None] * nchunk
    writes = [[] for _ in range(nchunk)]

    def start_read(c):
        r = c * CHUNK
        reads[c] = pltpu.make_async_copy(
            table_hbm.at[pl.ds(r, CHUNK), :], bufs[c % NBUF], rsems[c % NBUF]
        )
        reads[c].start()

    def start_writes(c):
        r = c * CHUNK
        for b in range(B):
            d = pltpu.make_async_copy(
                bufs[c % NBUF], out_hbm.at[b, pl.ds(r, CHUNK), :], wsems[c % NBUF]
            )
            d.start()
            writes[c].append(d)

    for c in range(min(NBUF, nchunk)):
        start_read(c)
    for c in range(nchunk):
        reads[c].wait()
        start_writes(c)
        nxt = c + NBUF
        if nxt < nchunk:
            for d in writes[c]:
                d.wait()
            start_read(nxt)
    for c in range(max(0, nchunk - NBUF), nchunk):
        for d in writes[c]:
            d.wait()


def kernel(position_ids, position_embeddings):
    Bd, S, H = position_ids.shape
    out = pl.pallas_call(
        _dma_body,
        in_specs=[pl.BlockSpec(memory_space=pltpu.HBM)],
        out_specs=pl.BlockSpec(memory_space=pltpu.HBM),
        out_shape=jax.ShapeDtypeStruct((Bd, S, H), jnp.float32),
        scratch_shapes=[
            pltpu.VMEM((CHUNK, H), jnp.float32),
            pltpu.VMEM((CHUNK, H), jnp.float32),
            pltpu.SemaphoreType.DMA,
            pltpu.SemaphoreType.DMA,
            pltpu.SemaphoreType.DMA,
            pltpu.SemaphoreType.DMA,
        ],
    )(position_embeddings[:S])
    return out
